# Initial kernel scaffold; baseline (speedup 1.0000x reference)
import jax
import jax.numpy as jnp
from jax.experimental import pallas as pl


def kernel(x1, edge_index1, edge_weight1, x2, edge_index2, edge_weight2, W1, b1, W2, b2, a1, a2, Wc, bc):
    raise NotImplementedError("write your pallas kernel here")


# speedup vs baseline: 10.6867x; 10.6867x over previous
import functools

import jax
import jax.numpy as jnp
from jax import lax
from jax.experimental import pallas as pl
from jax.experimental.pallas import tpu as pltpu
from jax.experimental.pallas import tpu_sc as plsc

N = 10000
F = 128
H = 256
E = 320000
NT = 16
C = 128
K = -(-(E // NT) // C)
EPT = K * C
RPT = 640
NPAD = NT * RPT
RB = 1000
GRID = N // RB
FH = F // 2


@functools.cache
def _build_sc_kernels():
    mesh = plsc.VectorSubcoreMesh(core_axis_name="c", subcore_axis_name="s",
                                  num_cores=2, num_subcores=NT)

    @functools.partial(
        pl.kernel,
        out_type=jax.ShapeDtypeStruct((2, NPAD), jnp.float32),
        mesh=mesh,
        scratch_types=[
            pltpu.VMEM((K, C), jnp.int32),
            pltpu.VMEM((K, C), jnp.float32),
            pltpu.VMEM((RPT,), jnp.float32),
            pltpu.VMEM_SHARED((NPAD,), jnp.float32),
        ],
    )
    def _deg_kernel(col1, ew1, col2, ew2, deg_out, colv, ewv, zv, dacc):
        cid = lax.axis_index("c")
        sid = lax.axis_index("s")

        def zero_body(i, carry):
            zv[pl.ds(i * 16, 16)] = jnp.zeros((16,), jnp.float32)
            return carry
        lax.fori_loop(0, RPT // 16, zero_body, 0)
        pltpu.sync_copy(zv, dacc.at[pl.ds(sid * RPT, RPT)])
        plsc.subcore_barrier()

        @pl.when(cid == 0)
        def _():
            pltpu.sync_copy(col1.at[sid], colv)
            pltpu.sync_copy(ew1.at[sid], ewv)

        @pl.when(cid == 1)
        def _():
            pltpu.sync_copy(col2.at[sid], colv)
            pltpu.sync_copy(ew2.at[sid], ewv)

        def chunk_body(j, carry):
            pltpu.sync_copy(ewv.at[j], dacc.at[colv.at[j]], add=True)
            return carry
        lax.fori_loop(0, K, chunk_body, 0)
        plsc.subcore_barrier()
        pltpu.sync_copy(dacc.at[pl.ds(sid * RPT, RPT)],
                        deg_out.at[cid, pl.ds(sid * RPT, RPT)])

    @functools.partial(
        pl.kernel,
        out_type=jax.ShapeDtypeStruct((2, 2, NPAD, FH), jnp.float32),
        mesh=mesh,
        scratch_types=[
            pltpu.VMEM((K, C), jnp.int32),
            pltpu.VMEM((K, C), jnp.int32),
            pltpu.VMEM((K, C), jnp.float32),
            pltpu.VMEM((C, FH), jnp.float32),
            pltpu.VMEM((64, FH), jnp.float32),
            pltpu.VMEM_SHARED((NPAD, FH), jnp.float32),
        ],
        compiler_params=pltpu.CompilerParams(use_tc_tiling_on_sc=False),
    )
    def _msg_kernel(xs1a, xs1b, xs2a, xs2b, row1, col1, ew1, row2, col2, ew2,
                    t_out, rowv, colv, ewv, gbuf, zbuf, tacc):
        cid = lax.axis_index("c")
        sid = lax.axis_index("s")

        def zero_row(r, carry):
            for k in range(FH // 16):
                zbuf[r, pl.ds(k * 16, 16)] = jnp.zeros((16,), jnp.float32)
            return carry
        lax.fori_loop(0, 64, zero_row, 0)

        @pl.when(cid == 0)
        def _():
            pltpu.sync_copy(row1.at[sid], rowv)
            pltpu.sync_copy(col1.at[sid], colv)
            pltpu.sync_copy(ew1.at[sid], ewv)

        @pl.when(cid == 1)
        def _():
            pltpu.sync_copy(row2.at[sid], rowv)
            pltpu.sync_copy(col2.at[sid], colv)
            pltpu.sync_copy(ew2.at[sid], ewv)

        def run_half(xs, h):
            def zero_copy(b, carry):
                pltpu.sync_copy(zbuf, tacc.at[pl.ds(sid * RPT + b * 64, 64)])
                return carry
            lax.fori_loop(0, RPT // 64, zero_copy, 0)
            plsc.subcore_barrier()

            def chunk_body(j, carry):
                pltpu.sync_copy(xs.at[rowv.at[j]], gbuf)

                def scale_body(i16, c2):
                    base = i16 * 16
                    ew16 = ewv[j, pl.ds(base, 16)]
                    for l in range(16):
                        s = ew16[l]
                        r = base + l
                        for k in range(FH // 16):
                            gbuf[r, pl.ds(k * 16, 16)] = (
                                gbuf[r, pl.ds(k * 16, 16)] * s)
                    return c2
                lax.fori_loop(0, C // 16, scale_body, 0)
                pltpu.sync_copy(gbuf, tacc.at[colv.at[j]], add=True)
                return carry
            lax.fori_loop(0, K, chunk_body, 0)
            plsc.subcore_barrier()
            pltpu.sync_copy(tacc.at[pl.ds(sid * RPT, RPT)],
                            t_out.at[cid, h, pl.ds(sid * RPT, RPT)])

        @pl.when(cid == 0)
        def _():
            run_half(xs1a, 0)
            run_half(xs1b, 1)

        @pl.when(cid == 1)
        def _():
            run_half(xs2a, 0)
            run_half(xs2b, 1)

    return _deg_kernel, _msg_kernel


def _scale_body(x1_ref, d1_ref, x2_ref, d2_ref,
                xs1a_ref, xs1b_ref, xs2a_ref, xs2b_ref):
    d1 = d1_ref[...] + 1.0
    xs1 = x1_ref[...] * jnp.where(d1 > 0, lax.rsqrt(d1), 0.0)
    xs1a_ref[...] = xs1[:, :FH]
    xs1b_ref[...] = xs1[:, FH:]
    d2 = d2_ref[...] + 1.0
    xs2 = x2_ref[...] * jnp.where(d2 > 0, lax.rsqrt(d2), 0.0)
    xs2a_ref[...] = xs2[:, :FH]
    xs2b_ref[...] = xs2[:, FH:]


_row = lambda i: (i, 0)
_fix = lambda i: (0, 0)

_scale_call = pl.pallas_call(
    _scale_body,
    grid=(GRID,),
    in_specs=[
        pl.BlockSpec((RB, F), _row),
        pl.BlockSpec((RB, 1), _row),
        pl.BlockSpec((RB, F), _row),
        pl.BlockSpec((RB, 1), _row),
    ],
    out_specs=[pl.BlockSpec((RB, FH), _row)] * 4,
    out_shape=[jax.ShapeDtypeStruct((N, FH), jnp.float32)] * 4,
)


def _dense_body(xs1a_ref, t1a_ref, xs1b_ref, t1b_ref, d1_ref,
                xs2a_ref, t2a_ref, xs2b_ref, t2b_ref, d2_ref,
                W1a_ref, W1b_ref, b1_ref, a1_ref,
                W2a_ref, W2b_ref, b2_ref, a2_ref,
                WcA_ref, WcB_ref, bc_ref, out_ref):
    d1 = d1_ref[...] + 1.0
    dis1 = jnp.where(d1 > 0, lax.rsqrt(d1), 0.0)
    z1 = (jnp.dot(dis1 * (xs1a_ref[...] + t1a_ref[...]), W1a_ref[...],
                  preferred_element_type=jnp.float32)
          + jnp.dot(dis1 * (xs1b_ref[...] + t1b_ref[...]), W1b_ref[...],
                    preferred_element_type=jnp.float32)
          + b1_ref[...])
    h1 = jnp.where(z1 >= 0.0, z1, a1_ref[...] * z1)

    d2 = d2_ref[...] + 1.0
    dis2 = jnp.where(d2 > 0, lax.rsqrt(d2), 0.0)
    z2 = (jnp.dot(dis2 * (xs2a_ref[...] + t2a_ref[...]), W2a_ref[...],
                  preferred_element_type=jnp.float32)
          + jnp.dot(dis2 * (xs2b_ref[...] + t2b_ref[...]), W2b_ref[...],
                    preferred_element_type=jnp.float32)
          + b2_ref[...])
    h2 = jnp.where(z2 >= 0.0, z2, a2_ref[...] * z2)

    z = (jnp.dot(h1, WcA_ref[...], preferred_element_type=jnp.float32)
         + jnp.dot(h2, WcB_ref[...], preferred_element_type=jnp.float32)
         + bc_ref[...])
    out_ref[...] = jnp.maximum(z, 0.0)


_dense_call = pl.pallas_call(
    _dense_body,
    grid=(GRID,),
    in_specs=(
        [pl.BlockSpec((RB, FH), _row)] * 4 + [pl.BlockSpec((RB, 1), _row)]
        + [pl.BlockSpec((RB, FH), _row)] * 4 + [pl.BlockSpec((RB, 1), _row)]
        + [pl.BlockSpec((FH, H), _fix), pl.BlockSpec((FH, H), _fix),
           pl.BlockSpec((1, H), _fix), pl.BlockSpec((1, H), _fix)]
        + [pl.BlockSpec((FH, H), _fix), pl.BlockSpec((FH, H), _fix),
           pl.BlockSpec((1, H), _fix), pl.BlockSpec((1, H), _fix)]
        + [pl.BlockSpec((H, H), _fix), pl.BlockSpec((H, H), _fix),
           pl.BlockSpec((1, H), _fix)]
    ),
    out_specs=pl.BlockSpec((RB, H), _row),
    out_shape=jax.ShapeDtypeStruct((N, H), jnp.float32),
)


def _prep_edges(edge_index, edge_weight):
    row = edge_index[0].astype(jnp.int32)
    col = edge_index[1].astype(jnp.int32)
    pad = NT * EPT - E
    row = jnp.concatenate([row, jnp.zeros((pad,), jnp.int32)]).reshape(NT, K, C)
    col = jnp.concatenate([col, jnp.zeros((pad,), jnp.int32)]).reshape(NT, K, C)
    ew = jnp.concatenate(
        [edge_weight, jnp.zeros((pad,), jnp.float32)]).reshape(NT, K, C)
    return row, col, ew


def kernel(x1, edge_index1, edge_weight1, x2, edge_index2, edge_weight2,
           W1, b1, W2, b2, a1, a2, Wc, bc):
    deg_kernel, msg_kernel = _build_sc_kernels()

    r1, c1, w1 = _prep_edges(edge_index1, edge_weight1)
    r2, c2, w2 = _prep_edges(edge_index2, edge_weight2)

    deg = deg_kernel(c1, w1, c2, w2)
    d1 = deg[0, :N].reshape(N, 1)
    d2 = deg[1, :N].reshape(N, 1)

    xs1a, xs1b, xs2a, xs2b = _scale_call(x1, d1, x2, d2)

    t = msg_kernel(xs1a, xs1b, xs2a, xs2b, r1, c1, w1, r2, c2, w2)

    return _dense_call(
        xs1a, t[0, 0, :N], xs1b, t[0, 1, :N], d1,
        xs2a, t[1, 0, :N], xs2b, t[1, 1, :N], d2,
        W1[:FH], W1[FH:], b1.reshape(1, H), a1.reshape(1, H),
        W2[:FH], W2[FH:], b2.reshape(1, H), a2.reshape(1, H),
        Wc[:H], Wc[H:], bc.reshape(1, H),
    )
